# baseline (speedup 1.0000x reference)
import functools

import jax
import jax.numpy as jnp
from jax.experimental import pallas as pl
from jax.experimental.pallas import tpu as pltpu

_EPS = 1e-12


def _prep_kernel(c_ref, cn_ref, cb_ref):
    c = c_ref[...]
    cn = c * jax.lax.rsqrt(
        jnp.maximum(jnp.sum(c * c, axis=1, keepdims=True), _EPS * _EPS)
    )
    cn_ref[...] = cn.astype(jnp.bfloat16)
    cb_ref[...] = c.astype(jnp.bfloat16)


def _centroid_kernel(x_ref, cn_ref, cb_ref, o_ref):
    xb = x_ref[...]
    xn = xb * jax.lax.rsqrt(
        jnp.maximum(jnp.sum(xb * xb, axis=1, keepdims=True), _EPS * _EPS)
    )

    sims = jax.lax.dot_general(
        xn.astype(jnp.bfloat16),
        cn_ref[...],
        (((1,), (1,)), ((), ())),
        preferred_element_type=jnp.float32,
    )
    e = jnp.exp(sims)
    s = jnp.sum(e, axis=1, keepdims=True)
    ctx = jnp.dot(
        e.astype(jnp.bfloat16), cb_ref[...], preferred_element_type=jnp.float32
    )
    o_ref[...] = ctx / s


@functools.partial(jax.jit, static_argnames=("block_n",))
def _centroid_layer(x, centroid_emb, block_n=512):
    n, d = x.shape
    p, _ = centroid_emb.shape

    cn_b, cb_b = pl.pallas_call(
        _prep_kernel,
        out_shape=[
            jax.ShapeDtypeStruct((p, d), jnp.bfloat16),
            jax.ShapeDtypeStruct((p, d), jnp.bfloat16),
        ],
    )(centroid_emb)

    return pl.pallas_call(
        _centroid_kernel,
        grid=(n // block_n,),
        in_specs=[
            pl.BlockSpec((block_n, d), lambda i: (i, 0)),
            pl.BlockSpec((p, d), lambda i: (0, 0)),
            pl.BlockSpec((p, d), lambda i: (0, 0)),
        ],
        out_specs=pl.BlockSpec((block_n, d), lambda i: (i, 0)),
        out_shape=jax.ShapeDtypeStruct((n, d), jnp.float32),
        compiler_params=pltpu.CompilerParams(
            dimension_semantics=("parallel",)
        ),
    )(x, cn_b, cb_b)


def kernel(x, centroid_emb):
    return _centroid_layer(x, centroid_emb)


# speedup vs baseline: 1.4002x; 1.4002x over previous
import functools

import jax
import jax.numpy as jnp
from jax.experimental import pallas as pl
from jax.experimental.pallas import tpu as pltpu

_EPS = 1e-12


def _centroid_kernel(x_ref, c_ref, o_ref, cn_ref, cb_ref):
    @pl.when(pl.program_id(0) == 0)
    def _():
        c = c_ref[...]
        cn = c * jax.lax.rsqrt(
            jnp.maximum(jnp.sum(c * c, axis=1, keepdims=True), _EPS * _EPS)
        )
        cn_ref[...] = cn.astype(jnp.bfloat16)
        cb_ref[...] = c.astype(jnp.bfloat16)

    xb = x_ref[...]
    xn = xb * jax.lax.rsqrt(
        jnp.maximum(jnp.sum(xb * xb, axis=1, keepdims=True), _EPS * _EPS)
    )

    sims = jax.lax.dot_general(
        xn.astype(jnp.bfloat16),
        cn_ref[...],
        (((1,), (1,)), ((), ())),
        preferred_element_type=jnp.float32,
    )
    e = jnp.exp(sims)
    s = jnp.sum(e, axis=1, keepdims=True)
    ctx = jnp.dot(
        e.astype(jnp.bfloat16), cb_ref[...], preferred_element_type=jnp.float32
    )
    o_ref[...] = ctx / s


@functools.partial(jax.jit, static_argnames=("block_n",))
def _centroid_layer(x, centroid_emb, block_n=1024):
    n, d = x.shape
    p, _ = centroid_emb.shape
    return pl.pallas_call(
        _centroid_kernel,
        grid=(n // block_n,),
        in_specs=[
            pl.BlockSpec((block_n, d), lambda i: (i, 0)),
            pl.BlockSpec((p, d), lambda i: (0, 0)),
        ],
        out_specs=pl.BlockSpec((block_n, d), lambda i: (i, 0)),
        out_shape=jax.ShapeDtypeStruct((n, d), jnp.float32),
        scratch_shapes=[
            pltpu.VMEM((p, d), jnp.bfloat16),
            pltpu.VMEM((p, d), jnp.bfloat16),
        ],
    )(x, centroid_emb)


def kernel(x, centroid_emb):
    return _centroid_layer(x, centroid_emb)


# speedup vs baseline: 1.6161x; 1.1542x over previous
import functools

import jax
import jax.numpy as jnp
from jax.experimental import pallas as pl
from jax.experimental.pallas import tpu as pltpu

_EPS = 1e-12


def _centroid_kernel(x_ref, c_ref, o_ref, cn_ref, cb_ref):
    @pl.when(pl.program_id(0) == 0)
    def _():
        c = c_ref[...]
        cn = c * jax.lax.rsqrt(
            jnp.maximum(jnp.sum(c * c, axis=1, keepdims=True), _EPS * _EPS)
        )
        cn_ref[...] = cn.astype(jnp.bfloat16)
        cb_ref[...] = c.astype(jnp.bfloat16)

    xb = x_ref[...]
    xn = xb * jax.lax.rsqrt(
        jnp.maximum(jnp.sum(xb * xb, axis=1, keepdims=True), _EPS * _EPS)
    )

    sims = jax.lax.dot_general(
        xn.astype(jnp.bfloat16),
        cn_ref[...],
        (((1,), (1,)), ((), ())),
        preferred_element_type=jnp.float32,
    )
    e = jnp.exp(sims)
    s = jnp.sum(e, axis=1, keepdims=True)
    ctx = jnp.dot(
        e.astype(jnp.bfloat16), cb_ref[...], preferred_element_type=jnp.float32
    )
    o_ref[...] = ctx / s


@functools.partial(jax.jit, static_argnames=("block_n",))
def _centroid_layer(x, centroid_emb, block_n=2048):
    n, d = x.shape
    p, _ = centroid_emb.shape
    return pl.pallas_call(
        _centroid_kernel,
        grid=(n // block_n,),
        in_specs=[
            pl.BlockSpec((block_n, d), lambda i: (i, 0)),
            pl.BlockSpec((p, d), lambda i: (0, 0)),
        ],
        out_specs=pl.BlockSpec((block_n, d), lambda i: (i, 0)),
        out_shape=jax.ShapeDtypeStruct((n, d), jnp.float32),
        scratch_shapes=[
            pltpu.VMEM((p, d), jnp.bfloat16),
            pltpu.VMEM((p, d), jnp.bfloat16),
        ],
    )(x, centroid_emb)


def kernel(x, centroid_emb):
    return _centroid_layer(x, centroid_emb)
